# baseline (speedup 1.0000x reference)
import functools

import numpy as np
import jax
import jax.numpy as jnp
from jax.experimental import pallas as pl
from jax.experimental.pallas import tpu as pltpu
from jax.experimental.pallas import tpu_sc as plsc

F = 26
B = 16384
VOCAB = 100000
D = 32
H1, H2 = 512, 256
FD = F * D
FF = F * F
GW = 128
BBLK = 512


def _sc_gather(tables2d, gidx, bc):
    mesh = plsc.VectorSubcoreMesh(core_axis_name="core", subcore_axis_name="subcore")

    @functools.partial(
        pl.kernel,
        out_type=jax.ShapeDtypeStruct((bc, FD), jnp.float32),
        mesh=mesh,
        compiler_params=pltpu.CompilerParams(use_tc_tiling_on_sc=False),
    )
    def gather_kernel(tab_hbm, idx_hbm, out_hbm):
        def body(i_vmem, o_vmem):
            pltpu.sync_copy(tab_hbm.at[i_vmem.at[0]], o_vmem)

        pltpu.emit_pipeline(
            body,
            grid=(F, bc // GW),
            in_specs=[
                pl.BlockSpec((1, GW), index_map=lambda f, b: (0, f * (bc // GW) + b))
            ],
            out_specs=[pl.BlockSpec((GW, D), index_map=lambda f, b: (b, f))],
            core_axis_name=("core", "subcore"),
            dimension_semantics=(pltpu.PARALLEL, pltpu.PARALLEL),
        )(idx_hbm, out_hbm)

    return gather_kernel(tables2d, gidx)


def _tc_body(flat_ref, w1a_ref, w1c_ref, b1_ref, w2_ref, b2_ref, w3_ref, b3_ref,
             out_ref, dots_ref):
    fl = flat_ref[...]
    ft = fl.T
    ft3 = ft.reshape(F, D, BBLK)
    for i in range(F):
        prod = ft3 * ft3[i][None]
        dots_ref[pl.ds(i * F, F), :] = jnp.sum(prod, axis=1)
    h = jnp.dot(w1a_ref[...], ft, preferred_element_type=jnp.float32)
    h = h + jnp.dot(w1c_ref[...], dots_ref[...], preferred_element_type=jnp.float32)
    h = jnp.maximum(h + b1_ref[...], 0.0)
    h2 = jnp.dot(w2_ref[...], h, preferred_element_type=jnp.float32) + b2_ref[...]
    h2 = jnp.maximum(h2, 0.0)
    out_ref[...] = jnp.dot(w3_ref[...], h2, preferred_element_type=jnp.float32) + b3_ref[...]


def _tc_mlp(embs, w1aT, w1cT, b1c, w2T, b2c, w3T, b3s, bc):
    return pl.pallas_call(
        _tc_body,
        grid=(bc // BBLK,),
        in_specs=[
            pl.BlockSpec((BBLK, FD), lambda i: (i, 0)),
            pl.BlockSpec((H1, FD), lambda i: (0, 0)),
            pl.BlockSpec((H1, FF), lambda i: (0, 0)),
            pl.BlockSpec((H1, 1), lambda i: (0, 0)),
            pl.BlockSpec((H2, H1), lambda i: (0, 0)),
            pl.BlockSpec((H2, 1), lambda i: (0, 0)),
            pl.BlockSpec((1, H2), lambda i: (0, 0)),
            pl.BlockSpec((1, 1), lambda i: (0, 0)),
        ],
        out_specs=pl.BlockSpec((1, BBLK), lambda i: (0, i)),
        out_shape=jax.ShapeDtypeStruct((1, bc), jnp.float32),
        scratch_shapes=[pltpu.VMEM((FF, BBLK), jnp.float32)],
    )(embs, w1aT, w1cT, b1c, w2T, b2c, w3T, b3s)


_IU, _JU = np.triu_indices(F, k=1)
_UP = np.asarray(_IU * F + _JU)
_LO = np.asarray(_JU * F + _IU)


NCHUNK = 4


def kernel(indices, tables, W1, b1, W2, b2, W3, b3):
    tables2d = tables.reshape(F * VOCAB, D)
    offs = (jnp.arange(F, dtype=jnp.int32) * VOCAB)[:, None]
    gall = indices + offs

    W1a = W1[:FD]
    W1b = 0.5 * W1[FD:]
    W1c = jnp.zeros((FF, H1), W1.dtype).at[_UP].set(W1b).at[_LO].set(W1b)

    bc = B // NCHUNK
    outs = []
    for c in range(NCHUNK):
        gidx = gall[:, c * bc:(c + 1) * bc].reshape(1, F * bc)
        flat = _sc_gather(tables2d, gidx, bc)
        outs.append(_tc_mlp(flat, W1a.T, W1c.T, b1[:, None], W2.T, b2[:, None],
                            W3.T, b3.reshape(1, 1), bc))
    return jnp.concatenate(outs, axis=1).reshape(B)


# speedup vs baseline: 1.0048x; 1.0048x over previous
import functools

import numpy as np
import jax
import jax.numpy as jnp
from jax.experimental import pallas as pl
from jax.experimental.pallas import tpu as pltpu
from jax.experimental.pallas import tpu_sc as plsc

F = 26
B = 16384
VOCAB = 100000
D = 32
H1, H2 = 512, 256
FD = F * D
FF = F * F
GW = 128
BBLK = 512


def _sc_gather(tables2d, gidx, bc):
    mesh = plsc.VectorSubcoreMesh(core_axis_name="core", subcore_axis_name="subcore")

    @functools.partial(
        pl.kernel,
        out_type=jax.ShapeDtypeStruct((bc, FD), jnp.float32),
        mesh=mesh,
        compiler_params=pltpu.CompilerParams(use_tc_tiling_on_sc=False),
    )
    def gather_kernel(tab_hbm, idx_hbm, out_hbm):
        def body(i_vmem, o_vmem):
            pltpu.sync_copy(tab_hbm.at[i_vmem.at[0]], o_vmem)

        pltpu.emit_pipeline(
            body,
            grid=(F, bc // GW),
            in_specs=[
                pl.BlockSpec((1, GW), index_map=lambda f, b: (0, f * (bc // GW) + b))
            ],
            out_specs=[pl.BlockSpec((GW, D), index_map=lambda f, b: (b, f))],
            core_axis_name=("core", "subcore"),
            dimension_semantics=(pltpu.PARALLEL, pltpu.PARALLEL),
        )(idx_hbm, out_hbm)

    return gather_kernel(tables2d, gidx)


def _tc_body(flat_ref, w1a_ref, w1c_ref, b1_ref, w2_ref, b2_ref, w3_ref, b3_ref,
             out_ref, dots_ref):
    fl = flat_ref[...]
    ft = fl.T
    ft3 = ft.reshape(F, D, BBLK)
    for i in range(F):
        prod = ft3 * ft3[i][None]
        dots_ref[pl.ds(i * F, F), :] = jnp.sum(prod, axis=1)
    h = jnp.dot(w1a_ref[...], ft, preferred_element_type=jnp.float32)
    h = h + jnp.dot(w1c_ref[...], dots_ref[...], preferred_element_type=jnp.float32)
    h = jnp.maximum(h + b1_ref[...], 0.0)
    h2 = jnp.dot(w2_ref[...], h, preferred_element_type=jnp.float32) + b2_ref[...]
    h2 = jnp.maximum(h2, 0.0)
    out_ref[...] = jnp.dot(w3_ref[...], h2, preferred_element_type=jnp.float32) + b3_ref[...]


def _tc_mlp(embs, w1aT, w1cT, b1c, w2T, b2c, w3T, b3s, bc):
    return pl.pallas_call(
        _tc_body,
        grid=(bc // BBLK,),
        in_specs=[
            pl.BlockSpec((BBLK, FD), lambda i: (i, 0)),
            pl.BlockSpec((H1, FD), lambda i: (0, 0)),
            pl.BlockSpec((H1, FF), lambda i: (0, 0)),
            pl.BlockSpec((H1, 1), lambda i: (0, 0)),
            pl.BlockSpec((H2, H1), lambda i: (0, 0)),
            pl.BlockSpec((H2, 1), lambda i: (0, 0)),
            pl.BlockSpec((1, H2), lambda i: (0, 0)),
            pl.BlockSpec((1, 1), lambda i: (0, 0)),
        ],
        out_specs=pl.BlockSpec((1, BBLK), lambda i: (0, i)),
        out_shape=jax.ShapeDtypeStruct((1, bc), jnp.float32),
        scratch_shapes=[pltpu.VMEM((FF, BBLK), jnp.float32)],
    )(embs, w1aT, w1cT, b1c, w2T, b2c, w3T, b3s)


_IU, _JU = np.triu_indices(F, k=1)
_UP = np.asarray(_IU * F + _JU)
_LO = np.asarray(_JU * F + _IU)


NCHUNK = 2


def kernel(indices, tables, W1, b1, W2, b2, W3, b3):
    tables2d = tables.reshape(F * VOCAB, D)
    offs = (jnp.arange(F, dtype=jnp.int32) * VOCAB)[:, None]
    gall = indices + offs

    W1a = W1[:FD]
    W1b = 0.5 * W1[FD:]
    W1c = jnp.zeros((FF, H1), W1.dtype).at[_UP].set(W1b).at[_LO].set(W1b)

    bc = B // NCHUNK
    outs = []
    for c in range(NCHUNK):
        gidx = gall[:, c * bc:(c + 1) * bc].reshape(1, F * bc)
        flat = _sc_gather(tables2d, gidx, bc)
        outs.append(_tc_mlp(flat, W1a.T, W1c.T, b1[:, None], W2.T, b2[:, None],
                            W3.T, b3.reshape(1, 1), bc))
    return jnp.concatenate(outs, axis=1).reshape(B)
